# baseline (speedup 1.0000x reference)
import functools

import jax
import jax.numpy as jnp
from jax import lax
from jax.experimental import pallas as pl
from jax.experimental.pallas import tpu as pltpu
from jax.experimental.pallas import tpu_sc as plsc

WINDOW = 5
LANES = 128


@functools.cache
def _make_prep_stage(B, L, V, D):
    def prep_body(ids_ref, tab_ref, w_ref, b_ref,
                  idsimg_ref, tabpad_ref, wt_ref, b2_ref):
        idsimg_ref[...] = jnp.concatenate(
            [ids_ref[...], jnp.zeros((B, LANES - L), jnp.int32)], axis=1)
        tabpad_ref[...] = jnp.concatenate(
            [tab_ref[...], jnp.zeros((V, LANES - D), jnp.float32)], axis=1)
        wt = jnp.transpose(w_ref[...])
        wt_ref[...] = jnp.concatenate(
            [wt, jnp.zeros((LANES - D, V), jnp.float32)], axis=0)
        b2_ref[...] = jnp.broadcast_to(b_ref[...].reshape(1, V), (8, V))

    return pl.pallas_call(
        prep_body,
        out_shape=[
            jax.ShapeDtypeStruct((B, LANES), jnp.int32),
            jax.ShapeDtypeStruct((V, LANES), jnp.float32),
            jax.ShapeDtypeStruct((LANES, V), jnp.float32),
            jax.ShapeDtypeStruct((8, V), jnp.float32),
        ],
    )


@functools.cache
def _make_sc_stage(B, L, V, D):
    info = plsc.get_sparse_core_info()
    NC, NS = info.num_cores, info.num_subcores
    NW = NC * NS
    NL = info.num_lanes
    assert B % NW == 0 and D == NL and L >= NL
    seq_per_w = B // NW
    rows_per_w = seq_per_w * L
    koffs = [i * NL for i in range(L // NL)]
    if L % NL:
        koffs.append(L - NL)
    SEQ_CHUNK = 8
    assert seq_per_w % SEQ_CHUNK == 0
    n_out_ch = seq_per_w // SEQ_CHUNK
    grp_rows = SEQ_CHUNK * L

    mesh = plsc.VectorSubcoreMesh(core_axis_name="c", subcore_axis_name="s")

    @functools.partial(
        pl.kernel,
        mesh=mesh,
        compiler_params=pltpu.CompilerParams(use_tc_tiling_on_sc=False,
                                             needs_layout_passes=False),
        out_type=jax.ShapeDtypeStruct((B * L, LANES), jnp.float32),
        scratch_types=[
            pltpu.VMEM((seq_per_w, LANES), jnp.int32),
            pltpu.VMEM((V, D), jnp.float32),
            pltpu.VMEM((rows_per_w * D,), jnp.float32),
            pltpu.VMEM((L + 1, D), jnp.float32),
            pltpu.VMEM((grp_rows, LANES), jnp.float32),
            pltpu.SemaphoreType.DMA,
        ],
    )
    def sc_kernel(ids_hbm, table_hbm, out_hbm, idx_v, tab_v, rows_v, cum_v,
                  grp_v, sem):
        wid = lax.axis_index("s") * NC + lax.axis_index("c")
        base = wid * rows_per_w

        pltpu.sync_copy(ids_hbm.at[pl.ds(wid * seq_per_w, seq_per_w)], idx_v)
        pltpu.sync_copy(table_hbm.at[pl.ds(0, V), pl.ds(0, D)], tab_v)

        lane16 = jnp.arange(NL, dtype=jnp.int32) * D
        dvecs = [jnp.full((NL,), d, jnp.int32) for d in range(D)]
        zero = jnp.zeros((D,), jnp.float32)

        def zero_body(t, carry):
            for k in range(LANES // D):
                grp_v[t, pl.ds(k * D, D)] = zero
            return carry

        lax.fori_loop(0, grp_rows, zero_body, 0)

        def seq_body(s8, c):
            s = c * SEQ_CHUNK + s8
            lrow0 = s8 * L
            fbase = s * (L * D)

            for koff in koffs:
                v = idx_v[s, pl.ds(koff, NL)]
                sbase = lane16 + (fbase + koff * D)
                for d in range(D):
                    val = plsc.load_gather(tab_v, [v, dvecs[d]])
                    plsc.store_scatter(rows_v, [sbase + d], val)

            cum_v[0, :] = zero

            def cum_body(t, acc):
                acc = acc + rows_v[pl.ds(fbase + t * D, D)]
                cum_v[t + 1, :] = acc
                return acc

            lax.fori_loop(0, L, cum_body, zero)

            def out_body(t, carry):
                hi = jnp.where(t + WINDOW > L, L - 1, t + WINDOW)
                lo = jnp.where(t < WINDOW, 1, t - WINDOW)
                interior = jnp.logical_and(t >= WINDOW, t + WINDOW <= L)
                cmask = jnp.where(interior, 1.0, 0.0).astype(jnp.float32)
                g = (cum_v[hi, :] - cum_v[lo, :]
                     - cmask * rows_v[pl.ds(fbase + t * D, D)])
                grp_v[lrow0 + t, pl.ds(0, D)] = g
                return carry

            lax.fori_loop(0, L, out_body, 0)
            return c

        for c in range(n_out_ch):
            lax.fori_loop(0, SEQ_CHUNK, seq_body, c)
            pltpu.sync_copy(grp_v, out_hbm.at[pl.ds(base + c * grp_rows,
                                                    grp_rows)])

    return sc_kernel


@functools.cache
def _make_tc_stage(M, V, MB=1024):
    assert M % MB == 0

    def mm_body(x_ref, w_ref, b_ref, o_ref):
        o_ref[...] = (
            lax.dot_general(
                x_ref[...], w_ref[...],
                (((1,), (0,)), ((), ())),
                preferred_element_type=jnp.float32,
            )
            + b_ref[0:1, :]
        )

    return pl.pallas_call(
        mm_body,
        grid=(M // MB,),
        in_specs=[
            pl.BlockSpec((MB, LANES), lambda i: (i, 0)),
            pl.BlockSpec((LANES, V), lambda i: (0, 0)),
            pl.BlockSpec((8, V), lambda i: (0, 0)),
        ],
        out_specs=pl.BlockSpec((MB, V), lambda i: (i, 0)),
        out_shape=jax.ShapeDtypeStruct((M, V), jnp.float32),
    )


def kernel(ids, emb_table, W, b):
    B, L = ids.shape
    V, D = emb_table.shape
    ids = ids.astype(jnp.int32)
    ids_img, tab_pad, wt, b2 = _make_prep_stage(B, L, V, D)(
        ids, emb_table, W, b)
    grouped = _make_sc_stage(B, L, V, D)(ids_img, tab_pad)
    out = _make_tc_stage(B * L, V)(grouped, wt, b2)
    return out.reshape(B, L, V)


# speedup vs baseline: 3.1307x; 3.1307x over previous
import functools

import jax
import jax.numpy as jnp
from jax import lax
from jax.experimental import pallas as pl
from jax.experimental.pallas import tpu as pltpu
from jax.experimental.pallas import tpu_sc as plsc

WINDOW = 5
LANES = 128


@functools.cache
def _make_prep_stage(B, L, V, D):
    def prep_body(ids_ref, tab_ref, b_ref, idsimg_ref, tabpad_ref, bt_ref):
        idsimg_ref[...] = jnp.concatenate(
            [ids_ref[...], jnp.zeros((B, LANES - L), jnp.int32)], axis=1)
        tabpad_ref[...] = jnp.concatenate(
            [tab_ref[...], jnp.zeros((V, LANES - D), jnp.float32)], axis=1)
        bt_ref[...] = jnp.transpose(
            jnp.broadcast_to(b_ref[...].reshape(1, V), (8, V)))

    return pl.pallas_call(
        prep_body,
        out_shape=[
            jax.ShapeDtypeStruct((B, LANES), jnp.int32),
            jax.ShapeDtypeStruct((V, LANES), jnp.float32),
            jax.ShapeDtypeStruct((V, 8), jnp.float32),
        ],
    )


@functools.cache
def _make_sc_stage(B, L, V, D):
    info = plsc.get_sparse_core_info()
    NC, NS = info.num_cores, info.num_subcores
    NW = NC * NS
    NL = info.num_lanes
    assert B % NW == 0 and D == NL and L >= NL
    seq_per_w = B // NW
    koffs = [i * NL for i in range(L // NL)]
    if L % NL:
        koffs.append(L - NL)

    mesh = plsc.VectorSubcoreMesh(core_axis_name="c", subcore_axis_name="s")

    @functools.partial(
        pl.kernel,
        mesh=mesh,
        compiler_params=pltpu.CompilerParams(use_tc_tiling_on_sc=False,
                                             needs_layout_passes=False),
        out_type=jax.ShapeDtypeStruct((L, D, B), jnp.float32),
        scratch_types=[
            pltpu.VMEM((seq_per_w, LANES), jnp.int32),
            pltpu.VMEM((V, D), jnp.float32),
            pltpu.VMEM((seq_per_w * L * D,), jnp.float32),
            pltpu.VMEM((L + 1, D), jnp.float32),
            pltpu.VMEM((L, D, seq_per_w), jnp.float32),
            pltpu.SemaphoreType.DMA,
        ],
    )
    def sc_kernel(ids_hbm, table_hbm, out_hbm, idx_v, tab_v, rows_v, cum_v,
                  gvt_v, sem):
        wid = lax.axis_index("s") * NC + lax.axis_index("c")

        pltpu.sync_copy(ids_hbm.at[pl.ds(wid * seq_per_w, seq_per_w)], idx_v)
        pltpu.sync_copy(table_hbm.at[pl.ds(0, V), pl.ds(0, D)], tab_v)

        lane16 = jnp.arange(NL, dtype=jnp.int32) * D
        dlane = jnp.arange(NL, dtype=jnp.int32)
        zeros_i = jnp.zeros((NL,), jnp.int32)
        dvecs = [jnp.full((NL,), d, jnp.int32) for d in range(D)]
        zero = jnp.zeros((D,), jnp.float32)

        def seq_body(s, carry):
            fbase = s * (L * D)

            for koff in koffs:
                v = idx_v[s, pl.ds(koff, NL)]
                sbase = lane16 + (fbase + koff * D)
                for d in range(D):
                    val = plsc.load_gather(tab_v, [v, dvecs[d]])
                    plsc.store_scatter(rows_v, [sbase + d], val)

            cum_v[0, :] = zero

            def cum_body(t, acc):
                acc = acc + rows_v[pl.ds(fbase + t * D, D)]
                cum_v[t + 1, :] = acc
                return acc

            lax.fori_loop(0, L, cum_body, zero)

            svec = zeros_i + s

            def out_body(t, carry2):
                hi = jnp.where(t + WINDOW > L, L - 1, t + WINDOW)
                lo = jnp.where(t < WINDOW, 1, t - WINDOW)
                interior = jnp.logical_and(t >= WINDOW, t + WINDOW <= L)
                cmask = jnp.where(interior, 1.0, 0.0).astype(jnp.float32)
                g = (cum_v[hi, :] - cum_v[lo, :]
                     - cmask * rows_v[pl.ds(fbase + t * D, D)])
                tvec = zeros_i + t
                plsc.store_scatter(gvt_v, [tvec, dlane, svec], g)
                return carry2

            lax.fori_loop(0, L, out_body, 0)
            return carry

        lax.fori_loop(0, seq_per_w, seq_body, 0)

        copies = []
        for l in range(L):
            copies.append(
                pltpu.async_copy(
                    gvt_v.at[l],
                    out_hbm.at[l, pl.ds(0, D),
                               pl.ds(wid * seq_per_w, seq_per_w)],
                    sem,
                )
            )
        for cp in copies:
            cp.wait()

    return sc_kernel


@functools.cache
def _make_tc_stage(B, L, V, D, LB=1):
    assert L % LB == 0

    def mm_body(x_ref, w_ref, b_ref, o_ref):
        w = w_ref[...]
        bias = b_ref[...][:, 0:1]
        for j in range(LB):
            o_ref[j] = (
                lax.dot_general(
                    w, x_ref[j],
                    (((1,), (0,)), ((), ())),
                    preferred_element_type=jnp.float32,
                )
                + bias
            )

    return pl.pallas_call(
        mm_body,
        grid=(L // LB,),
        in_specs=[
            pl.BlockSpec((LB, D, B), lambda i: (i, 0, 0)),
            pl.BlockSpec((V, D), lambda i: (0, 0)),
            pl.BlockSpec((V, 8), lambda i: (0, 0)),
        ],
        out_specs=pl.BlockSpec((LB, V, B), lambda i: (i, 0, 0)),
        out_shape=jax.ShapeDtypeStruct((L, V, B), jnp.float32),
    )


def kernel(ids, emb_table, W, b):
    B, L = ids.shape
    V, D = emb_table.shape
    ids = ids.astype(jnp.int32)
    ids_img, tab_pad, b_t = _make_prep_stage(B, L, V, D)(ids, emb_table, b)
    grouped_t = _make_sc_stage(B, L, V, D)(ids_img, tab_pad)
    out_t = _make_tc_stage(B, L, V, D)(grouped_t, W, b_t)
    return jnp.transpose(out_t, (2, 0, 1))


# speedup vs baseline: 3.4544x; 1.1034x over previous
import functools

import jax
import jax.numpy as jnp
from jax import lax
from jax.experimental import pallas as pl
from jax.experimental.pallas import tpu as pltpu
from jax.experimental.pallas import tpu_sc as plsc

WINDOW = 5
LANES = 128


@functools.cache
def _make_prep_stage(B, L, V, D):
    def prep_body(ids_ref, tab_ref, b_ref, idsimg_ref, tabpad_ref, bt_ref):
        idsimg_ref[...] = jnp.concatenate(
            [ids_ref[...], jnp.zeros((B, LANES - L), jnp.int32)], axis=1)
        tabpad_ref[...] = jnp.concatenate(
            [tab_ref[...], jnp.zeros((V, LANES - D), jnp.float32)], axis=1)
        bt_ref[...] = jnp.transpose(
            jnp.broadcast_to(b_ref[...].reshape(1, V), (8, V)))

    return pl.pallas_call(
        prep_body,
        out_shape=[
            jax.ShapeDtypeStruct((B, LANES), jnp.int32),
            jax.ShapeDtypeStruct((V, LANES), jnp.float32),
            jax.ShapeDtypeStruct((V, 8), jnp.float32),
        ],
    )


@functools.cache
def _make_sc_stage(B, L, V, D):
    info = plsc.get_sparse_core_info()
    NC, NS = info.num_cores, info.num_subcores
    NW = NC * NS
    NL = info.num_lanes
    assert B % NW == 0 and D == NL and L >= NL
    seq_per_w = B // NW
    koffs = [i * NL for i in range(L // NL)]
    if L % NL:
        koffs.append(L - NL)

    mesh = plsc.VectorSubcoreMesh(core_axis_name="c", subcore_axis_name="s")

    @functools.partial(
        pl.kernel,
        mesh=mesh,
        compiler_params=pltpu.CompilerParams(use_tc_tiling_on_sc=False,
                                             needs_layout_passes=False),
        out_type=jax.ShapeDtypeStruct((L, D, B), jnp.float32),
        scratch_types=[
            pltpu.VMEM((seq_per_w, LANES), jnp.int32),
            pltpu.VMEM((V, D), jnp.float32),
            pltpu.VMEM((seq_per_w * L * D,), jnp.float32),
            pltpu.VMEM((L, D, seq_per_w), jnp.float32),
            pltpu.SemaphoreType.DMA,
        ],
    )
    def sc_kernel(ids_hbm, table_hbm, out_hbm, idx_v, tab_v, rows_v,
                  gvt_v, sem):
        wid = lax.axis_index("s") * NC + lax.axis_index("c")

        pltpu.sync_copy(ids_hbm.at[pl.ds(wid * seq_per_w, seq_per_w)], idx_v)
        pltpu.sync_copy(table_hbm.at[pl.ds(0, V), pl.ds(0, D)], tab_v)

        lane16 = jnp.arange(NL, dtype=jnp.int32) * D
        dlane = jnp.arange(NL, dtype=jnp.int32)
        zeros_i = jnp.zeros((NL,), jnp.int32)
        dvecs = [jnp.full((NL,), d, jnp.int32) for d in range(D)]
        tvecs = [jnp.full((NL,), t, jnp.int32) for t in range(L)]
        zero = jnp.zeros((D,), jnp.float32)

        def seq_body(s, carry):
            fbase = s * (L * D)
            svec = zeros_i + s

            for koff in koffs:
                v = idx_v[s, pl.ds(koff, NL)]
                sbase = lane16 + (fbase + koff * D)
                for d in range(D):
                    val = plsc.load_gather(tab_v, [v, dvecs[d]])
                    plsc.store_scatter(rows_v, [sbase + d], val)

            es, cums = [], [zero]
            for t in range(L):
                e = rows_v[pl.ds(fbase + t * D, D)]
                es.append(e)
                cums.append(cums[-1] + e)
            for t in range(L):
                if t < WINDOW:
                    g = cums[t + WINDOW] - cums[1]
                elif t + WINDOW > L:
                    g = cums[L - 1] - cums[t - WINDOW]
                else:
                    g = cums[t + WINDOW] - cums[t - WINDOW] - es[t]
                plsc.store_scatter(gvt_v, [tvecs[t], dlane, svec], g)
            return carry

        lax.fori_loop(0, seq_per_w, seq_body, 0)

        copies = []
        for l in range(L):
            copies.append(
                pltpu.async_copy(
                    gvt_v.at[l],
                    out_hbm.at[l, pl.ds(0, D),
                               pl.ds(wid * seq_per_w, seq_per_w)],
                    sem,
                )
            )
        for cp in copies:
            cp.wait()

    return sc_kernel


@functools.cache
def _make_tc_stage(B, L, V, D, LB=1):
    assert L % LB == 0

    def mm_body(x_ref, w_ref, b_ref, o_ref):
        w = w_ref[...]
        bias = b_ref[...][:, 0:1]
        for j in range(LB):
            o_ref[j] = (
                lax.dot_general(
                    w, x_ref[j],
                    (((1,), (0,)), ((), ())),
                    preferred_element_type=jnp.float32,
                )
                + bias
            )

    return pl.pallas_call(
        mm_body,
        grid=(L // LB,),
        in_specs=[
            pl.BlockSpec((LB, D, B), lambda i: (i, 0, 0)),
            pl.BlockSpec((V, D), lambda i: (0, 0)),
            pl.BlockSpec((V, 8), lambda i: (0, 0)),
        ],
        out_specs=pl.BlockSpec((LB, V, B), lambda i: (i, 0, 0)),
        out_shape=jax.ShapeDtypeStruct((L, V, B), jnp.float32),
    )


def kernel(ids, emb_table, W, b):
    B, L = ids.shape
    V, D = emb_table.shape
    ids = ids.astype(jnp.int32)
    ids_img, tab_pad, b_t = _make_prep_stage(B, L, V, D)(ids, emb_table, b)
    grouped_t = _make_sc_stage(B, L, V, D)(ids_img, tab_pad)
    out_t = _make_tc_stage(B, L, V, D)(grouped_t, W, b_t)
    return jnp.transpose(out_t, (2, 0, 1))


# speedup vs baseline: 3.8790x; 1.1229x over previous
import functools

import jax
import jax.numpy as jnp
from jax import lax
from jax.experimental import pallas as pl
from jax.experimental.pallas import tpu as pltpu
from jax.experimental.pallas import tpu_sc as plsc

WINDOW = 5
LANES = 128


@functools.cache
def _make_prep_stage(B, L, V, D):
    def prep_body(ids_ref, tab_ref, b_ref, idsimg_ref, tabpad_ref, bt_ref):
        idsimg_ref[...] = jnp.concatenate(
            [ids_ref[...], jnp.zeros((B, LANES - L), jnp.int32)], axis=1)
        tabpad_ref[...] = jnp.concatenate(
            [tab_ref[...], jnp.zeros((V, LANES - D), jnp.float32)], axis=1)
        bt_ref[...] = jnp.transpose(
            jnp.broadcast_to(b_ref[...].reshape(1, V), (8, V)))

    return pl.pallas_call(
        prep_body,
        out_shape=[
            jax.ShapeDtypeStruct((B, LANES), jnp.int32),
            jax.ShapeDtypeStruct((V, LANES), jnp.float32),
            jax.ShapeDtypeStruct((V, 8), jnp.float32),
        ],
    )


@functools.cache
def _make_sc_stage(B, L, V, D):
    info = plsc.get_sparse_core_info()
    NC, NS = info.num_cores, info.num_subcores
    NW = NC * NS
    NL = info.num_lanes
    assert B % NW == 0 and D == NL and L >= NL
    seq_per_w = B // NW
    koffs = [i * NL for i in range(L // NL)]
    if L % NL:
        koffs.append(L - NL)

    mesh = plsc.VectorSubcoreMesh(core_axis_name="c", subcore_axis_name="s")

    @functools.partial(
        pl.kernel,
        mesh=mesh,
        compiler_params=pltpu.CompilerParams(use_tc_tiling_on_sc=False,
                                             needs_layout_passes=False),
        out_type=jax.ShapeDtypeStruct((L, D, B), jnp.float32),
        scratch_types=[
            pltpu.VMEM((seq_per_w, LANES), jnp.int32),
            pltpu.VMEM((V, D), jnp.float32),
            pltpu.VMEM((seq_per_w * L, D), jnp.float32),
            pltpu.VMEM((L * D, seq_per_w), jnp.float32),
            pltpu.SemaphoreType.DMA,
        ],
    )
    def sc_kernel(ids_hbm, table_hbm, out_hbm, idx_v, tab_v, rows_v,
                  gvt_v, sem):
        wid = lax.axis_index("s") * NC + lax.axis_index("c")

        pltpu.sync_copy(ids_hbm.at[pl.ds(wid * seq_per_w, seq_per_w)], idx_v)
        pltpu.sync_copy(table_hbm.at[pl.ds(0, V), pl.ds(0, D)], tab_v)

        lane_i = jnp.arange(NL, dtype=jnp.int32)
        zeros_i = jnp.zeros((NL,), jnp.int32)
        dvecs = [jnp.full((NL,), d, jnp.int32) for d in range(D)]
        zero = jnp.zeros((D,), jnp.float32)
        PAIR = 2

        def seq_body(i, carry):
            seqs = [i * PAIR + j for j in range(PAIR)]
            svecs = [zeros_i + s for s in seqs]

            for s in seqs:
                for koff in koffs:
                    v = idx_v[s, pl.ds(koff, NL)]
                    tokvec = lane_i + (s * L + koff)
                    vals = [plsc.load_gather(tab_v, [v, dvecs[d]])
                            for d in range(D)]
                    for d in range(D):
                        plsc.store_scatter(rows_v, [tokvec, dvecs[d]],
                                           vals[d])

            es = [{} for _ in seqs]
            cums = [{0: zero} for _ in seqs]

            def emit(j, p):
                c = cums[j]
                if p < WINDOW:
                    g = c[p + WINDOW] - c[1]
                elif p + WINDOW > L:
                    g = c[L - 1] - c[p - WINDOW]
                else:
                    g = c[p + WINDOW] - c[p - WINDOW] - es[j][p]
                rowvec = lane_i + p * D
                plsc.store_scatter(gvt_v, [rowvec, svecs[j]], g)

            lag = WINDOW + 1
            for t in range(L):
                for j, s in enumerate(seqs):
                    e = rows_v[s * L + t, :]
                    es[j][t] = e
                    cums[j][t + 1] = cums[j][t] + e
                for j in range(PAIR):
                    p = t - lag
                    if p >= 0:
                        emit(j, p)
            for p in range(L - lag, L):
                for j in range(PAIR):
                    emit(j, p)
            return carry

        lax.fori_loop(0, seq_per_w // PAIR, seq_body, 0)

        copies = []
        for l in range(L):
            copies.append(
                pltpu.async_copy(
                    gvt_v.at[pl.ds(l * D, D)],
                    out_hbm.at[l, pl.ds(0, D),
                               pl.ds(wid * seq_per_w, seq_per_w)],
                    sem,
                )
            )
        for cp in copies:
            cp.wait()

    return sc_kernel


@functools.cache
def _make_tc_stage(B, L, V, D, LB=1):
    assert L % LB == 0

    def mm_body(x_ref, w_ref, b_ref, o_ref):
        w = w_ref[...]
        bias = b_ref[...][:, 0:1]
        for j in range(LB):
            o_ref[j] = (
                lax.dot_general(
                    w, x_ref[j],
                    (((1,), (0,)), ((), ())),
                    preferred_element_type=jnp.float32,
                )
                + bias
            )

    return pl.pallas_call(
        mm_body,
        grid=(L // LB,),
        in_specs=[
            pl.BlockSpec((LB, D, B), lambda i: (i, 0, 0)),
            pl.BlockSpec((V, D), lambda i: (0, 0)),
            pl.BlockSpec((V, 8), lambda i: (0, 0)),
        ],
        out_specs=pl.BlockSpec((LB, V, B), lambda i: (i, 0, 0)),
        out_shape=jax.ShapeDtypeStruct((L, V, B), jnp.float32),
    )


def kernel(ids, emb_table, W, b):
    B, L = ids.shape
    V, D = emb_table.shape
    ids = ids.astype(jnp.int32)
    ids_img, tab_pad, b_t = _make_prep_stage(B, L, V, D)(ids, emb_table, b)
    grouped_t = _make_sc_stage(B, L, V, D)(ids_img, tab_pad)
    out_t = _make_tc_stage(B, L, V, D)(grouped_t, W, b_t)
    return jnp.transpose(out_t, (2, 0, 1))


# speedup vs baseline: 4.0826x; 1.0525x over previous
import functools

import jax
import jax.numpy as jnp
from jax import lax
from jax.experimental import pallas as pl
from jax.experimental.pallas import tpu as pltpu
from jax.experimental.pallas import tpu_sc as plsc

WINDOW = 5
LANES = 128


@functools.cache
def _make_prep_stage(B, L, V, D):
    def prep_body(ids_ref, tab_ref, b_ref, idsimg_ref, tabpad_ref, bt_ref):
        idsimg_ref[...] = jnp.concatenate(
            [ids_ref[...], jnp.zeros((B, LANES - L), jnp.int32)], axis=1)
        tabpad_ref[...] = jnp.concatenate(
            [tab_ref[...], jnp.zeros((V, LANES - D), jnp.float32)], axis=1)
        bt_ref[...] = jnp.transpose(
            jnp.broadcast_to(b_ref[...].reshape(1, V), (8, V)))

    return pl.pallas_call(
        prep_body,
        out_shape=[
            jax.ShapeDtypeStruct((B, LANES), jnp.int32),
            jax.ShapeDtypeStruct((V, LANES), jnp.float32),
            jax.ShapeDtypeStruct((V, 8), jnp.float32),
        ],
    )


@functools.cache
def _make_sc_stage(B, L, V, D):
    info = plsc.get_sparse_core_info()
    NC, NS = info.num_cores, info.num_subcores
    NW = NC * NS
    NL = info.num_lanes
    assert B % NW == 0 and D == NL and L >= NL
    seq_per_w = B // NW
    koffs = [i * NL for i in range(L // NL)]
    if L % NL:
        koffs.append(L - NL)

    mesh = plsc.VectorSubcoreMesh(core_axis_name="c", subcore_axis_name="s")

    @functools.partial(
        pl.kernel,
        mesh=mesh,
        compiler_params=pltpu.CompilerParams(use_tc_tiling_on_sc=False,
                                             needs_layout_passes=False),
        out_type=jax.ShapeDtypeStruct((L, D, B), jnp.float32),
        scratch_types=[
            pltpu.VMEM((seq_per_w, LANES), jnp.int32),
            pltpu.VMEM((V, D), jnp.float32),
            pltpu.VMEM((seq_per_w * L, D), jnp.float32),
            pltpu.VMEM((L * D, seq_per_w), jnp.float32),
            pltpu.SemaphoreType.DMA,
        ],
    )
    def sc_kernel(ids_hbm, table_hbm, out_hbm, idx_v, tab_v, rows_v,
                  gvt_v, sem):
        wid = lax.axis_index("s") * NC + lax.axis_index("c")

        pltpu.sync_copy(ids_hbm.at[pl.ds(wid * seq_per_w, seq_per_w)], idx_v)
        pltpu.sync_copy(table_hbm.at[pl.ds(0, V), pl.ds(0, D)], tab_v)

        lane_i = jnp.arange(NL, dtype=jnp.int32)
        zeros_i = jnp.zeros((NL,), jnp.int32)
        dvecs = [jnp.full((NL,), d, jnp.int32) for d in range(D)]
        zero = jnp.zeros((D,), jnp.float32)
        PAIR = 2

        def seq_body(i, carry):
            seqs = [i * PAIR + j for j in range(PAIR)]
            svecs = [zeros_i + s for s in seqs]

            for s in seqs:
                for koff in koffs:
                    v = idx_v[s, pl.ds(koff, NL)]
                    tokvec = lane_i + (s * L + koff)
                    vals = [plsc.load_gather(tab_v, [v, dvecs[d]])
                            for d in range(D)]
                    for d in range(D):
                        plsc.store_scatter(rows_v, [tokvec, dvecs[d]],
                                           vals[d])

            es = [{} for _ in seqs]
            cums = [{0: zero} for _ in seqs]

            def emit(j, p):
                c = cums[j]
                if p < WINDOW:
                    g = c[p + WINDOW] - c[1]
                elif p + WINDOW > L:
                    g = c[L - 1] - c[p - WINDOW]
                else:
                    g = c[p + WINDOW] - c[p - WINDOW] - es[j][p]
                rowvec = lane_i + p * D
                plsc.store_scatter(gvt_v, [rowvec, svecs[j]], g)

            lag = WINDOW + 1
            for t in range(L):
                for j, s in enumerate(seqs):
                    e = rows_v[s * L + t, :]
                    es[j][t] = e
                    cums[j][t + 1] = cums[j][t] + e
                for j in range(PAIR):
                    p = t - lag
                    if p >= 0:
                        emit(j, p)
            for p in range(L - lag, L):
                for j in range(PAIR):
                    emit(j, p)
            return carry

        lax.fori_loop(0, seq_per_w // PAIR, seq_body, 0)

        copies = []
        for l in range(L):
            copies.append(
                pltpu.async_copy(
                    gvt_v.at[pl.ds(l * D, D)],
                    out_hbm.at[l, pl.ds(0, D),
                               pl.ds(wid * seq_per_w, seq_per_w)],
                    sem,
                )
            )
        for cp in copies:
            cp.wait()

    return sc_kernel


@functools.cache
def _make_tc_stage(B, L, V, D, LB=2):
    assert L % LB == 0

    def mm_body(x_ref, w_ref, b_ref, o_ref):
        w = w_ref[...]
        bias = b_ref[...][:, 0:1]
        for j in range(LB):
            o_ref[j] = (
                lax.dot_general(
                    w, x_ref[j],
                    (((1,), (0,)), ((), ())),
                    preferred_element_type=jnp.float32,
                )
                + bias
            )

    return pl.pallas_call(
        mm_body,
        grid=(L // LB,),
        in_specs=[
            pl.BlockSpec((LB, D, B), lambda i: (i, 0, 0)),
            pl.BlockSpec((V, D), lambda i: (0, 0)),
            pl.BlockSpec((V, 8), lambda i: (0, 0)),
        ],
        out_specs=pl.BlockSpec((LB, V, B), lambda i: (i, 0, 0)),
        out_shape=jax.ShapeDtypeStruct((L, V, B), jnp.float32),
    )


def kernel(ids, emb_table, W, b):
    B, L = ids.shape
    V, D = emb_table.shape
    ids = ids.astype(jnp.int32)
    ids_img, tab_pad, b_t = _make_prep_stage(B, L, V, D)(ids, emb_table, b)
    grouped_t = _make_sc_stage(B, L, V, D)(ids_img, tab_pad)
    out_t = _make_tc_stage(B, L, V, D)(grouped_t, W, b_t)
    return jnp.transpose(out_t, (2, 0, 1))
